# baseline (speedup 1.0000x reference)
import functools

import jax
import jax.numpy as jnp
from jax import lax
from jax.experimental import pallas as pl
from jax.experimental.pallas import tpu as pltpu
from jax.experimental.pallas import tpu_sc as plsc

N = 10000
E = 320000
D = 128

NC = 2
NS = 16
NW = NC * NS

NP = 10240
RPS = NP // NS
EPW = E // NW
K = 40
NB = EPW // K
NBUF = 6
DNBUF = 5
ZR = 16

_mesh = plsc.VectorSubcoreMesh(core_axis_name="c", subcore_axis_name="s")


@functools.partial(
    pl.kernel,
    out_type=jax.ShapeDtypeStruct((NC, 2, NP), jnp.float32),
    mesh=_mesh,
    scratch_types=[
        pltpu.VMEM((NB, 2, K), jnp.int32),
        pltpu.VMEM((K,), jnp.float32),
        pltpu.VMEM((RPS,), jnp.float32),
        pltpu.VMEM_SHARED((NP,), jnp.float32),
        pltpu.VMEM_SHARED((NP,), jnp.float32),
        pltpu.SemaphoreType.DMA((DNBUF,)),
    ],
)
def _deg_kernel(ei_hbm, out_hbm, ei_all, ones_v, zb, do_sh, di_sh, sems):
    cid = lax.axis_index("c")
    sid = lax.axis_index("s")
    wid = cid * NS + sid

    def fill(i, _):
        zb[pl.ds(i * 16, 16)] = jnp.zeros((16,), jnp.float32)
        ones_v[pl.ds((i % (K // 16)) * 16, 16)] = jnp.ones((16,), jnp.float32)
        return 0

    lax.fori_loop(0, RPS // 16, fill, 0)
    pltpu.sync_copy(ei_hbm.at[wid], ei_all)
    pltpu.sync_copy(zb, do_sh.at[pl.ds(sid * RPS, RPS)])
    pltpu.sync_copy(zb, di_sh.at[pl.ds(sid * RPS, RPS)])
    plsc.subcore_barrier()

    for b in range(DNBUF - 1):
        pltpu.async_copy(ones_v, do_sh.at[ei_all.at[b, 0]], sems.at[b], add=True)
        pltpu.async_copy(ones_v, di_sh.at[ei_all.at[b, 1]], sems.at[b], add=True)

    def group(g, _):
        for b in range(DNBUF):
            i = g * DNBUF + b
            pltpu.make_async_copy(ones_v, do_sh.at[ei_all.at[i, 0]], sems.at[b]).wait()
            pltpu.make_async_copy(ones_v, di_sh.at[ei_all.at[i, 1]], sems.at[b]).wait()
            nxt = i + DNBUF - 1
            bx = (b + DNBUF - 1) % DNBUF

            @pl.when(nxt < NB)
            def _():
                pltpu.async_copy(ones_v, do_sh.at[ei_all.at[nxt, 0]], sems.at[bx], add=True)
                pltpu.async_copy(ones_v, di_sh.at[ei_all.at[nxt, 1]], sems.at[bx], add=True)

        return 0

    lax.fori_loop(0, NB // DNBUF, group, 0)
    plsc.subcore_barrier()

    sl = pl.ds(sid * RPS, RPS)
    pltpu.sync_copy(do_sh.at[sl], out_hbm.at[cid, 0, sl])
    pltpu.sync_copy(di_sh.at[sl], out_hbm.at[cid, 1, sl])


IB = 8


@functools.partial(
    pl.kernel,
    out_type=jax.ShapeDtypeStruct((NC, NP, D), jnp.float32),
    mesh=_mesh,
    scratch_types=[
        pltpu.VMEM((IB, 2, K), jnp.int32),
        pltpu.VMEM((NBUF, K, D), jnp.float32),
        pltpu.VMEM((ZR, D), jnp.float32),
        pltpu.VMEM_SHARED((NP, D), jnp.float32),
        pltpu.SemaphoreType.DMA((IB,)),
        pltpu.SemaphoreType.DMA((NBUF,)),
        pltpu.SemaphoreType.DMA((NBUF,)),
        pltpu.SemaphoreType.DMA,
    ],
)
def _agg_kernel(h_hbm, ei_hbm, out_hbm, ei_v, rows_v, zb, agg_sh, isems, gsems, ssems, csem):
    cid = lax.axis_index("c")
    sid = lax.axis_index("s")
    wid = cid * NS + sid

    def fill(t, _):
        zb[t // (D // 16), pl.ds((t % (D // 16)) * 16, 16)] = jnp.zeros(
            (16,), jnp.float32
        )
        return 0

    lax.fori_loop(0, ZR * (D // 16), fill, 0)

    def zero_fire(j, _):
        pltpu.async_copy(zb, agg_sh.at[pl.ds(sid * RPS + j * ZR, ZR)], csem)
        return 0

    def zero_drain(j, _):
        pltpu.make_async_copy(zb, agg_sh.at[pl.ds(sid * RPS, ZR)], csem).wait()
        return 0

    lax.fori_loop(0, RPS // ZR, zero_fire, 0)
    lax.fori_loop(0, RPS // ZR, zero_drain, 0)
    plsc.subcore_barrier()

    for q in range(IB):
        pltpu.async_copy(ei_hbm.at[wid, q], ei_v.at[q], isems.at[q])
    for b in range(NBUF - 1):
        pltpu.make_async_copy(ei_hbm.at[wid, b], ei_v.at[b], isems.at[b]).wait()
        pltpu.async_copy(h_hbm.at[ei_v.at[b, 0]], rows_v.at[b], gsems.at[b])

    GA = NBUF - 1
    RA = IB - 1

    def body(i, _):
        b = lax.rem(i, NBUF)
        qb = lax.rem(i, IB)
        pltpu.make_async_copy(h_hbm.at[ei_v.at[qb, 0]], rows_v.at[b], gsems.at[b]).wait()

        @pl.when(i >= 1)
        def _():
            bm = lax.rem(i + GA, NBUF)
            qm = lax.rem(i + RA, IB)
            pltpu.make_async_copy(
                rows_v.at[bm], agg_sh.at[ei_v.at[qm, 1]], ssems.at[bm]
            ).wait()

        @pl.when((i >= 1) & (i + RA < NB))
        def _():
            qm = lax.rem(i + RA, IB)
            pltpu.async_copy(ei_hbm.at[wid, i + RA], ei_v.at[qm], isems.at[qm])

        @pl.when(i + GA < NB)
        def _():
            q2 = lax.rem(i + GA, IB)
            r2 = lax.rem(i + GA, NBUF)
            pltpu.make_async_copy(ei_hbm.at[wid, i + GA], ei_v.at[q2], isems.at[q2]).wait()
            pltpu.async_copy(h_hbm.at[ei_v.at[q2, 0]], rows_v.at[r2], gsems.at[r2])

        pltpu.async_copy(rows_v.at[b], agg_sh.at[ei_v.at[qb, 1]], ssems.at[b], add=True)
        return 0

    lax.fori_loop(0, NB, body, 0)
    pltpu.make_async_copy(
        rows_v.at[(NB - 1) % NBUF],
        agg_sh.at[ei_v.at[(NB - 1) % IB, 1]],
        ssems.at[(NB - 1) % NBUF],
    ).wait()
    plsc.subcore_barrier()

    def out_fire(j, _):
        sl = pl.ds(sid * RPS + j * ZR, ZR)
        pltpu.async_copy(agg_sh.at[sl], out_hbm.at[cid, sl], csem)
        return 0

    def out_drain(j, _):
        sl = pl.ds(sid * RPS, ZR)
        pltpu.make_async_copy(agg_sh.at[sl], out_hbm.at[cid, sl], csem).wait()
        return 0

    lax.fori_loop(0, RPS // ZR, out_fire, 0)
    lax.fori_loop(0, RPS // ZR, out_drain, 0)


_RB = 512
_GRID = NP // _RB


def _ns_nd(d):
    outd = d[:, 0:1] + d[:, 2:3]
    ind = d[:, 1:2] + d[:, 3:4]
    ns = lax.rsqrt(jnp.maximum(outd, 1.0))
    nd = lax.rsqrt(jnp.maximum(ind, 1.0))
    return ns, nd


def _scale_body(f_ref, d_ref, o_ref):
    ns, _ = _ns_nd(d_ref[...])
    o_ref[...] = f_ref[...] * ns


def _layer1_body(p_ref, d_ref, w_ref, b_ref, o_ref):
    ns, nd = _ns_nd(d_ref[...])
    rst = (p_ref[0] + p_ref[1]) * nd
    h = jnp.dot(rst, w_ref[...], preferred_element_type=jnp.float32) + b_ref[...]
    o_ref[...] = jnp.maximum(h, 0.0) * ns


def _layer2_body(p_ref, d_ref, w_ref, b_ref, o_ref):
    _, nd = _ns_nd(d_ref[...])
    rst = (p_ref[0] + p_ref[1]) * nd
    o_ref[...] = jnp.dot(rst, w_ref[...], preferred_element_type=jnp.float32) + b_ref[...]


_scale = pl.pallas_call(
    _scale_body,
    grid=(_GRID,),
    in_specs=[
        pl.BlockSpec((_RB, D), lambda i: (i, 0)),
        pl.BlockSpec((_RB, 4), lambda i: (i, 0)),
    ],
    out_specs=pl.BlockSpec((_RB, D), lambda i: (i, 0)),
    out_shape=jax.ShapeDtypeStruct((NP, D), jnp.float32),
)

_layer1 = pl.pallas_call(
    _layer1_body,
    grid=(_GRID,),
    in_specs=[
        pl.BlockSpec((NC, _RB, D), lambda i: (0, i, 0)),
        pl.BlockSpec((_RB, 4), lambda i: (i, 0)),
        pl.BlockSpec((D, D), lambda i: (0, 0)),
        pl.BlockSpec((1, D), lambda i: (0, 0)),
    ],
    out_specs=pl.BlockSpec((_RB, D), lambda i: (i, 0)),
    out_shape=jax.ShapeDtypeStruct((NP, D), jnp.float32),
)

_layer2 = pl.pallas_call(
    _layer2_body,
    grid=(_GRID,),
    in_specs=[
        pl.BlockSpec((NC, _RB, D), lambda i: (0, i, 0)),
        pl.BlockSpec((_RB, 4), lambda i: (i, 0)),
        pl.BlockSpec((D, D), lambda i: (0, 0)),
        pl.BlockSpec((1, D), lambda i: (0, 0)),
    ],
    out_specs=pl.BlockSpec((_RB, D), lambda i: (i, 0)),
    out_shape=jax.ShapeDtypeStruct((NP, D), jnp.float32),
)


def kernel(feat, edge_index, W1, b1, W2, b2):
    ei = jnp.transpose(edge_index.reshape(2, NW, NB, K), (1, 2, 0, 3))

    degs = _deg_kernel(ei)
    degs_t = degs.reshape(2 * NC, NP).T

    featp = jnp.pad(feat, ((0, NP - N), (0, 0)))
    h1 = _scale(featp, degs_t)
    p1 = _agg_kernel(h1, ei)
    h2 = _layer1(p1, degs_t, W1, b1.reshape(1, D))
    p2 = _agg_kernel(h2, ei)
    out = _layer2(p2, degs_t, W2, b2.reshape(1, D))
    return out[:N]


# speedup vs baseline: 1.0669x; 1.0669x over previous
import functools

import jax
import jax.numpy as jnp
from jax import lax
from jax.experimental import pallas as pl
from jax.experimental.pallas import tpu as pltpu
from jax.experimental.pallas import tpu_sc as plsc

N = 10000
E = 320000
D = 128

NC = 2
NS = 16
NW = NC * NS

NP = 10240
RPS = NP // NS
EPW = E // NW
K = 80
NB = EPW // K
NBUF = 3
DNBUF = 5
ZR = 16

_mesh = plsc.VectorSubcoreMesh(core_axis_name="c", subcore_axis_name="s")


@functools.partial(
    pl.kernel,
    out_type=jax.ShapeDtypeStruct((NC, 2, NP), jnp.float32),
    mesh=_mesh,
    scratch_types=[
        pltpu.VMEM((NB, 2, K), jnp.int32),
        pltpu.VMEM((K,), jnp.float32),
        pltpu.VMEM((RPS,), jnp.float32),
        pltpu.VMEM_SHARED((NP,), jnp.float32),
        pltpu.VMEM_SHARED((NP,), jnp.float32),
        pltpu.SemaphoreType.DMA((DNBUF,)),
    ],
)
def _deg_kernel(ei_hbm, out_hbm, ei_all, ones_v, zb, do_sh, di_sh, sems):
    cid = lax.axis_index("c")
    sid = lax.axis_index("s")
    wid = cid * NS + sid

    def fill(i, _):
        zb[pl.ds(i * 16, 16)] = jnp.zeros((16,), jnp.float32)
        ones_v[pl.ds((i % (K // 16)) * 16, 16)] = jnp.ones((16,), jnp.float32)
        return 0

    lax.fori_loop(0, RPS // 16, fill, 0)
    pltpu.sync_copy(ei_hbm.at[wid], ei_all)
    pltpu.sync_copy(zb, do_sh.at[pl.ds(sid * RPS, RPS)])
    pltpu.sync_copy(zb, di_sh.at[pl.ds(sid * RPS, RPS)])
    plsc.subcore_barrier()

    for b in range(DNBUF - 1):
        pltpu.async_copy(ones_v, do_sh.at[ei_all.at[b, 0]], sems.at[b], add=True)
        pltpu.async_copy(ones_v, di_sh.at[ei_all.at[b, 1]], sems.at[b], add=True)

    def group(g, _):
        for b in range(DNBUF):
            i = g * DNBUF + b
            pltpu.make_async_copy(ones_v, do_sh.at[ei_all.at[i, 0]], sems.at[b]).wait()
            pltpu.make_async_copy(ones_v, di_sh.at[ei_all.at[i, 1]], sems.at[b]).wait()
            nxt = i + DNBUF - 1
            bx = (b + DNBUF - 1) % DNBUF

            @pl.when(nxt < NB)
            def _():
                pltpu.async_copy(ones_v, do_sh.at[ei_all.at[nxt, 0]], sems.at[bx], add=True)
                pltpu.async_copy(ones_v, di_sh.at[ei_all.at[nxt, 1]], sems.at[bx], add=True)

        return 0

    lax.fori_loop(0, NB // DNBUF, group, 0)
    plsc.subcore_barrier()

    sl = pl.ds(sid * RPS, RPS)
    pltpu.sync_copy(do_sh.at[sl], out_hbm.at[cid, 0, sl])
    pltpu.sync_copy(di_sh.at[sl], out_hbm.at[cid, 1, sl])


IB = 5


@functools.partial(
    pl.kernel,
    out_type=jax.ShapeDtypeStruct((NC, NP, D), jnp.float32),
    mesh=_mesh,
    scratch_types=[
        pltpu.VMEM((IB, 2, K), jnp.int32),
        pltpu.VMEM((NBUF, K, D), jnp.float32),
        pltpu.VMEM((ZR, D), jnp.float32),
        pltpu.VMEM_SHARED((NP, D), jnp.float32),
        pltpu.SemaphoreType.DMA((IB,)),
        pltpu.SemaphoreType.DMA((NBUF,)),
        pltpu.SemaphoreType.DMA((NBUF,)),
        pltpu.SemaphoreType.DMA,
    ],
)
def _agg_kernel(h_hbm, ei_hbm, out_hbm, ei_v, rows_v, zb, agg_sh, isems, gsems, ssems, csem):
    cid = lax.axis_index("c")
    sid = lax.axis_index("s")
    wid = cid * NS + sid

    def fill(t, _):
        zb[t // (D // 16), pl.ds((t % (D // 16)) * 16, 16)] = jnp.zeros(
            (16,), jnp.float32
        )
        return 0

    for q in range(IB):
        pltpu.async_copy(ei_hbm.at[wid, q], ei_v.at[q], isems.at[q])

    lax.fori_loop(0, ZR * (D // 16), fill, 0)

    def zero_fire(j, _):
        pltpu.async_copy(zb, agg_sh.at[pl.ds(sid * RPS + j * ZR, ZR)], csem)
        return 0

    def zero_drain(j, _):
        pltpu.make_async_copy(zb, agg_sh.at[pl.ds(sid * RPS, ZR)], csem).wait()
        return 0

    lax.fori_loop(0, RPS // ZR, zero_fire, 0)

    for b in range(NBUF - 1):
        pltpu.make_async_copy(ei_hbm.at[wid, b], ei_v.at[b], isems.at[b]).wait()
        pltpu.async_copy(h_hbm.at[ei_v.at[b, 0]], rows_v.at[b], gsems.at[b])

    lax.fori_loop(0, RPS // ZR, zero_drain, 0)
    plsc.subcore_barrier()

    def body(i, _):
        b = lax.rem(i, NBUF)
        qb = lax.rem(i, IB)
        pltpu.make_async_copy(h_hbm.at[ei_v.at[qb, 0]], rows_v.at[b], gsems.at[b]).wait()

        @pl.when(i >= 1)
        def _():
            bm = lax.rem(i + 2, NBUF)
            qm = lax.rem(i + 4, IB)
            pltpu.make_async_copy(
                rows_v.at[bm], agg_sh.at[ei_v.at[qm, 1]], ssems.at[bm]
            ).wait()

        @pl.when((i >= 1) & (i + 4 < NB))
        def _():
            qm = lax.rem(i + 4, IB)
            pltpu.async_copy(ei_hbm.at[wid, i + 4], ei_v.at[qm], isems.at[qm])

        @pl.when(i + 2 < NB)
        def _():
            q2 = lax.rem(i + 2, IB)
            r2 = lax.rem(i + 2, NBUF)
            pltpu.make_async_copy(ei_hbm.at[wid, i + 2], ei_v.at[q2], isems.at[q2]).wait()
            pltpu.async_copy(h_hbm.at[ei_v.at[q2, 0]], rows_v.at[r2], gsems.at[r2])

        pltpu.async_copy(rows_v.at[b], agg_sh.at[ei_v.at[qb, 1]], ssems.at[b], add=True)
        return 0

    lax.fori_loop(0, NB, body, 0)
    pltpu.make_async_copy(
        rows_v.at[(NB - 1) % NBUF],
        agg_sh.at[ei_v.at[(NB - 1) % IB, 1]],
        ssems.at[(NB - 1) % NBUF],
    ).wait()
    plsc.subcore_barrier()

    def out_fire(j, _):
        sl = pl.ds(sid * RPS + j * ZR, ZR)
        pltpu.async_copy(agg_sh.at[sl], out_hbm.at[cid, sl], csem)
        return 0

    def out_drain(j, _):
        sl = pl.ds(sid * RPS, ZR)
        pltpu.make_async_copy(agg_sh.at[sl], out_hbm.at[cid, sl], csem).wait()
        return 0

    lax.fori_loop(0, RPS // ZR, out_fire, 0)
    lax.fori_loop(0, RPS // ZR, out_drain, 0)


_RB = 512
_GRID = NP // _RB


def _ns_nd(d):
    outd = d[:, 0:1] + d[:, 2:3]
    ind = d[:, 1:2] + d[:, 3:4]
    ns = lax.rsqrt(jnp.maximum(outd, 1.0))
    nd = lax.rsqrt(jnp.maximum(ind, 1.0))
    return ns, nd


def _scale_body(f_ref, d_ref, o_ref):
    ns, _ = _ns_nd(d_ref[...])
    o_ref[...] = f_ref[...] * ns


def _layer1_body(p_ref, d_ref, w_ref, b_ref, o_ref):
    ns, nd = _ns_nd(d_ref[...])
    rst = (p_ref[0] + p_ref[1]) * nd
    h = jnp.dot(rst, w_ref[...], preferred_element_type=jnp.float32) + b_ref[...]
    o_ref[...] = jnp.maximum(h, 0.0) * ns


def _layer2_body(p_ref, d_ref, w_ref, b_ref, o_ref):
    _, nd = _ns_nd(d_ref[...])
    rst = (p_ref[0] + p_ref[1]) * nd
    o_ref[...] = jnp.dot(rst, w_ref[...], preferred_element_type=jnp.float32) + b_ref[...]


_scale = pl.pallas_call(
    _scale_body,
    grid=(_GRID,),
    in_specs=[
        pl.BlockSpec((_RB, D), lambda i: (i, 0)),
        pl.BlockSpec((_RB, 4), lambda i: (i, 0)),
    ],
    out_specs=pl.BlockSpec((_RB, D), lambda i: (i, 0)),
    out_shape=jax.ShapeDtypeStruct((N, D), jnp.float32),
)

_layer1 = pl.pallas_call(
    _layer1_body,
    grid=(_GRID,),
    in_specs=[
        pl.BlockSpec((NC, _RB, D), lambda i: (0, i, 0)),
        pl.BlockSpec((_RB, 4), lambda i: (i, 0)),
        pl.BlockSpec((D, D), lambda i: (0, 0)),
        pl.BlockSpec((1, D), lambda i: (0, 0)),
    ],
    out_specs=pl.BlockSpec((_RB, D), lambda i: (i, 0)),
    out_shape=jax.ShapeDtypeStruct((N, D), jnp.float32),
)

_layer2 = pl.pallas_call(
    _layer2_body,
    grid=(_GRID,),
    in_specs=[
        pl.BlockSpec((NC, _RB, D), lambda i: (0, i, 0)),
        pl.BlockSpec((_RB, 4), lambda i: (i, 0)),
        pl.BlockSpec((D, D), lambda i: (0, 0)),
        pl.BlockSpec((1, D), lambda i: (0, 0)),
    ],
    out_specs=pl.BlockSpec((_RB, D), lambda i: (i, 0)),
    out_shape=jax.ShapeDtypeStruct((N, D), jnp.float32),
)


def kernel(feat, edge_index, W1, b1, W2, b2):
    ei = jnp.transpose(edge_index.reshape(2, NW, NB, K), (1, 2, 0, 3))

    degs = _deg_kernel(ei)
    degs_t = degs.reshape(2 * NC, NP).T

    h1 = _scale(feat, degs_t)
    p1 = _agg_kernel(h1, ei)
    h2 = _layer1(p1, degs_t, W1, b1.reshape(1, D))
    p2 = _agg_kernel(h2, ei)
    return _layer2(p2, degs_t, W2, b2.reshape(1, D))


# speedup vs baseline: 1.1303x; 1.0594x over previous
import functools

import jax
import jax.numpy as jnp
from jax import lax
from jax.experimental import pallas as pl
from jax.experimental.pallas import tpu as pltpu
from jax.experimental.pallas import tpu_sc as plsc

N = 10000
E = 320000
D = 128

NC = 2
NS = 16
NW = NC * NS

NP = 10240
RPS = NP // NS
EPW = E // NW
K = 80
NB = EPW // K
NBUF = 3
DNBUF = 5
ZR = 16

_mesh = plsc.VectorSubcoreMesh(core_axis_name="c", subcore_axis_name="s")


@functools.partial(
    pl.kernel,
    out_type=jax.ShapeDtypeStruct((NC, 2, NP), jnp.float32),
    mesh=_mesh,
    scratch_types=[
        pltpu.VMEM((NB, 2, K), jnp.int32),
        pltpu.VMEM((K,), jnp.float32),
        pltpu.VMEM((RPS,), jnp.float32),
        pltpu.VMEM_SHARED((NP,), jnp.float32),
        pltpu.VMEM_SHARED((NP,), jnp.float32),
        pltpu.SemaphoreType.DMA((DNBUF,)),
    ],
)
def _deg_kernel(ei_hbm, out_hbm, ei_all, ones_v, zb, do_sh, di_sh, sems):
    cid = lax.axis_index("c")
    sid = lax.axis_index("s")
    wid = cid * NS + sid

    def fill(i, _):
        zb[pl.ds(i * 16, 16)] = jnp.zeros((16,), jnp.float32)
        ones_v[pl.ds((i % (K // 16)) * 16, 16)] = jnp.ones((16,), jnp.float32)
        return 0

    lax.fori_loop(0, RPS // 16, fill, 0)
    pltpu.sync_copy(ei_hbm.at[wid], ei_all)
    pltpu.sync_copy(zb, do_sh.at[pl.ds(sid * RPS, RPS)])
    pltpu.sync_copy(zb, di_sh.at[pl.ds(sid * RPS, RPS)])
    plsc.subcore_barrier()

    for b in range(DNBUF - 1):
        pltpu.async_copy(ones_v, do_sh.at[ei_all.at[b, 0]], sems.at[b], add=True)
        pltpu.async_copy(ones_v, di_sh.at[ei_all.at[b, 1]], sems.at[b], add=True)

    def group(g, _):
        for b in range(DNBUF):
            i = g * DNBUF + b
            pltpu.make_async_copy(ones_v, do_sh.at[ei_all.at[i, 0]], sems.at[b]).wait()
            pltpu.make_async_copy(ones_v, di_sh.at[ei_all.at[i, 1]], sems.at[b]).wait()
            nxt = i + DNBUF - 1
            bx = (b + DNBUF - 1) % DNBUF

            @pl.when(nxt < NB)
            def _():
                pltpu.async_copy(ones_v, do_sh.at[ei_all.at[nxt, 0]], sems.at[bx], add=True)
                pltpu.async_copy(ones_v, di_sh.at[ei_all.at[nxt, 1]], sems.at[bx], add=True)

        return 0

    lax.fori_loop(0, NB // DNBUF, group, 0)
    plsc.subcore_barrier()

    sl = pl.ds(sid * RPS, RPS)
    pltpu.sync_copy(do_sh.at[sl], out_hbm.at[cid, 0, sl])
    pltpu.sync_copy(di_sh.at[sl], out_hbm.at[cid, 1, sl])


IB = 5


@functools.partial(
    pl.kernel,
    out_type=jax.ShapeDtypeStruct((NC, NP, D), jnp.float32),
    mesh=_mesh,
    scratch_types=[
        pltpu.VMEM((IB, 2, K), jnp.int32),
        pltpu.VMEM((NBUF, K, D), jnp.float32),
        pltpu.VMEM((ZR, D), jnp.float32),
        pltpu.VMEM_SHARED((NP, D), jnp.float32),
        pltpu.SemaphoreType.DMA((IB,)),
        pltpu.SemaphoreType.DMA((NBUF,)),
        pltpu.SemaphoreType.DMA((NBUF,)),
        pltpu.SemaphoreType.DMA,
    ],
)
def _agg_kernel(h_hbm, ei_hbm, out_hbm, ei_v, rows_v, zb, agg_sh, isems, gsems, ssems, csem):
    cid = lax.axis_index("c")
    sid = lax.axis_index("s")
    wid = cid * NS + sid

    def fill(t, _):
        zb[t // (D // 16), pl.ds((t % (D // 16)) * 16, 16)] = jnp.zeros(
            (16,), jnp.float32
        )
        return 0

    for q in range(IB):
        pltpu.async_copy(ei_hbm.at[wid, q], ei_v.at[q], isems.at[q])

    lax.fori_loop(0, ZR * (D // 16), fill, 0)

    def zero_fire(j, _):
        pltpu.async_copy(zb, agg_sh.at[pl.ds(sid * RPS + j * ZR, ZR)], csem)
        return 0

    def zero_drain(j, _):
        pltpu.make_async_copy(zb, agg_sh.at[pl.ds(sid * RPS, ZR)], csem).wait()
        return 0

    lax.fori_loop(0, RPS // ZR, zero_fire, 0)

    for b in range(NBUF - 1):
        pltpu.make_async_copy(ei_hbm.at[wid, b], ei_v.at[b], isems.at[b]).wait()
        pltpu.async_copy(h_hbm.at[ei_v.at[b, 0]], rows_v.at[b], gsems.at[b])

    lax.fori_loop(0, RPS // ZR, zero_drain, 0)
    plsc.subcore_barrier()

    def body(i, _):
        b = lax.rem(i, NBUF)
        qb = lax.rem(i, IB)
        pltpu.make_async_copy(h_hbm.at[ei_v.at[qb, 0]], rows_v.at[b], gsems.at[b]).wait()

        @pl.when(i >= 1)
        def _():
            bm = lax.rem(i + 2, NBUF)
            qm = lax.rem(i + 4, IB)
            pltpu.make_async_copy(
                rows_v.at[bm], agg_sh.at[ei_v.at[qm, 1]], ssems.at[bm]
            ).wait()

        @pl.when((i >= 1) & (i + 4 < NB))
        def _():
            qm = lax.rem(i + 4, IB)
            pltpu.async_copy(ei_hbm.at[wid, i + 4], ei_v.at[qm], isems.at[qm])

        @pl.when(i + 2 < NB)
        def _():
            q2 = lax.rem(i + 2, IB)
            r2 = lax.rem(i + 2, NBUF)
            pltpu.make_async_copy(ei_hbm.at[wid, i + 2], ei_v.at[q2], isems.at[q2]).wait()
            pltpu.async_copy(h_hbm.at[ei_v.at[q2, 0]], rows_v.at[r2], gsems.at[r2])

        pltpu.async_copy(rows_v.at[b], agg_sh.at[ei_v.at[qb, 1]], ssems.at[b], add=True)
        return 0

    lax.fori_loop(0, NB, body, 0)
    pltpu.make_async_copy(
        rows_v.at[(NB - 1) % NBUF],
        agg_sh.at[ei_v.at[(NB - 1) % IB, 1]],
        ssems.at[(NB - 1) % NBUF],
    ).wait()
    plsc.subcore_barrier()

    def out_fire(j, _):
        sl = pl.ds(sid * RPS + j * ZR, ZR)
        pltpu.async_copy(agg_sh.at[sl], out_hbm.at[cid, sl], csem)
        return 0

    def out_drain(j, _):
        sl = pl.ds(sid * RPS, ZR)
        pltpu.make_async_copy(agg_sh.at[sl], out_hbm.at[cid, sl], csem).wait()
        return 0

    lax.fori_loop(0, RPS // ZR, out_fire, 0)
    lax.fori_loop(0, RPS // ZR, out_drain, 0)


_RB = 1024
_GRID = NP // _RB


def _ns_nd(d):
    outd = d[:, 0:1] + d[:, 2:3]
    ind = d[:, 1:2] + d[:, 3:4]
    ns = lax.rsqrt(jnp.maximum(outd, 1.0))
    nd = lax.rsqrt(jnp.maximum(ind, 1.0))
    return ns, nd


def _scale_body(f_ref, d_ref, o_ref):
    ns, _ = _ns_nd(d_ref[...])
    o_ref[...] = f_ref[...] * ns


def _layer1_body(p_ref, d_ref, w_ref, b_ref, o_ref):
    ns, nd = _ns_nd(d_ref[...])
    rst = (p_ref[0] + p_ref[1]) * nd
    h = jnp.dot(rst, w_ref[...], preferred_element_type=jnp.float32) + b_ref[...]
    o_ref[...] = jnp.maximum(h, 0.0) * ns


def _layer2_body(p_ref, d_ref, w_ref, b_ref, o_ref):
    _, nd = _ns_nd(d_ref[...])
    rst = (p_ref[0] + p_ref[1]) * nd
    o_ref[...] = jnp.dot(rst, w_ref[...], preferred_element_type=jnp.float32) + b_ref[...]


_scale = pl.pallas_call(
    _scale_body,
    grid=(_GRID,),
    in_specs=[
        pl.BlockSpec((_RB, D), lambda i: (i, 0)),
        pl.BlockSpec((_RB, 4), lambda i: (i, 0)),
    ],
    out_specs=pl.BlockSpec((_RB, D), lambda i: (i, 0)),
    out_shape=jax.ShapeDtypeStruct((N, D), jnp.float32),
)

_layer1 = pl.pallas_call(
    _layer1_body,
    grid=(_GRID,),
    in_specs=[
        pl.BlockSpec((NC, _RB, D), lambda i: (0, i, 0)),
        pl.BlockSpec((_RB, 4), lambda i: (i, 0)),
        pl.BlockSpec((D, D), lambda i: (0, 0)),
        pl.BlockSpec((1, D), lambda i: (0, 0)),
    ],
    out_specs=pl.BlockSpec((_RB, D), lambda i: (i, 0)),
    out_shape=jax.ShapeDtypeStruct((N, D), jnp.float32),
)

_layer2 = pl.pallas_call(
    _layer2_body,
    grid=(_GRID,),
    in_specs=[
        pl.BlockSpec((NC, _RB, D), lambda i: (0, i, 0)),
        pl.BlockSpec((_RB, 4), lambda i: (i, 0)),
        pl.BlockSpec((D, D), lambda i: (0, 0)),
        pl.BlockSpec((1, D), lambda i: (0, 0)),
    ],
    out_specs=pl.BlockSpec((_RB, D), lambda i: (i, 0)),
    out_shape=jax.ShapeDtypeStruct((N, D), jnp.float32),
)


def kernel(feat, edge_index, W1, b1, W2, b2):
    ei = jnp.transpose(edge_index.reshape(2, NW, NB, K), (1, 2, 0, 3))

    degs = _deg_kernel(ei)
    degs_t = degs.reshape(2 * NC, NP).T

    h1 = _scale(feat, degs_t)
    p1 = _agg_kernel(h1, ei)
    h2 = _layer1(p1, degs_t, W1, b1.reshape(1, D))
    p2 = _agg_kernel(h2, ei)
    return _layer2(p2, degs_t, W2, b2.reshape(1, D))


# speedup vs baseline: 1.1576x; 1.0242x over previous
import functools

import jax
import jax.numpy as jnp
from jax import lax
from jax.experimental import pallas as pl
from jax.experimental.pallas import tpu as pltpu
from jax.experimental.pallas import tpu_sc as plsc

N = 10000
E = 320000
D = 128

NC = 2
NS = 16
NW = NC * NS

NP = 10240
RPS = NP // NS
EPW = E // NW
K = 80
NB = EPW // K
NBUF = 3
DNBUF = 5
ZR = 16

_mesh = plsc.VectorSubcoreMesh(core_axis_name="c", subcore_axis_name="s")


@functools.partial(
    pl.kernel,
    out_type=jax.ShapeDtypeStruct((NC, 2, NP), jnp.float32),
    mesh=_mesh,
    scratch_types=[
        pltpu.VMEM((NB, 2, K), jnp.int32),
        pltpu.VMEM((K,), jnp.float32),
        pltpu.VMEM((RPS,), jnp.float32),
        pltpu.VMEM_SHARED((NP,), jnp.float32),
        pltpu.VMEM_SHARED((NP,), jnp.float32),
        pltpu.SemaphoreType.DMA((DNBUF,)),
    ],
)
def _deg_kernel(ei_hbm, out_hbm, ei_all, ones_v, zb, do_sh, di_sh, sems):
    cid = lax.axis_index("c")
    sid = lax.axis_index("s")
    wid = cid * NS + sid

    def fill(i, _):
        zb[pl.ds(i * 16, 16)] = jnp.zeros((16,), jnp.float32)
        ones_v[pl.ds((i % (K // 16)) * 16, 16)] = jnp.ones((16,), jnp.float32)
        return 0

    lax.fori_loop(0, RPS // 16, fill, 0)
    pltpu.sync_copy(ei_hbm.at[wid], ei_all)
    pltpu.sync_copy(zb, do_sh.at[pl.ds(sid * RPS, RPS)])
    pltpu.sync_copy(zb, di_sh.at[pl.ds(sid * RPS, RPS)])
    plsc.subcore_barrier()

    for b in range(DNBUF - 1):
        pltpu.async_copy(ones_v, do_sh.at[ei_all.at[b, 0]], sems.at[b], add=True)
        pltpu.async_copy(ones_v, di_sh.at[ei_all.at[b, 1]], sems.at[b], add=True)

    def group(g, _):
        for b in range(DNBUF):
            i = g * DNBUF + b
            pltpu.make_async_copy(ones_v, do_sh.at[ei_all.at[i, 0]], sems.at[b]).wait()
            pltpu.make_async_copy(ones_v, di_sh.at[ei_all.at[i, 1]], sems.at[b]).wait()
            nxt = i + DNBUF - 1
            bx = (b + DNBUF - 1) % DNBUF

            @pl.when(nxt < NB)
            def _():
                pltpu.async_copy(ones_v, do_sh.at[ei_all.at[nxt, 0]], sems.at[bx], add=True)
                pltpu.async_copy(ones_v, di_sh.at[ei_all.at[nxt, 1]], sems.at[bx], add=True)

        return 0

    lax.fori_loop(0, NB // DNBUF, group, 0)
    plsc.subcore_barrier()

    sl = pl.ds(sid * RPS, RPS)
    pltpu.sync_copy(do_sh.at[sl], out_hbm.at[cid, 0, sl])
    pltpu.sync_copy(di_sh.at[sl], out_hbm.at[cid, 1, sl])


IB = 5


@functools.partial(
    pl.kernel,
    out_type=jax.ShapeDtypeStruct((NC, NP, D), jnp.float32),
    mesh=_mesh,
    scratch_types=[
        pltpu.VMEM((IB, 2, K), jnp.int32),
        pltpu.VMEM((NBUF, K, D), jnp.float32),
        pltpu.VMEM((ZR, D), jnp.float32),
        pltpu.VMEM_SHARED((NP, D), jnp.float32),
        pltpu.SemaphoreType.DMA((IB,)),
        pltpu.SemaphoreType.DMA((NBUF,)),
        pltpu.SemaphoreType.DMA((NBUF,)),
        pltpu.SemaphoreType.DMA,
    ],
)
def _agg_kernel(h_hbm, ei_hbm, out_hbm, ei_v, rows_v, zb, agg_sh, isems, gsems, ssems, csem):
    cid = lax.axis_index("c")
    sid = lax.axis_index("s")
    wid = cid * NS + sid

    def fill(t, _):
        zb[t // (D // 16), pl.ds((t % (D // 16)) * 16, 16)] = jnp.zeros(
            (16,), jnp.float32
        )
        return 0

    for q in range(IB):
        pltpu.async_copy(ei_hbm.at[wid, q], ei_v.at[q], isems.at[q])

    lax.fori_loop(0, ZR * (D // 16), fill, 0)

    def zero_fire(j, _):
        pltpu.async_copy(zb, agg_sh.at[pl.ds(sid * RPS + j * ZR, ZR)], csem)
        return 0

    def zero_drain(j, _):
        pltpu.make_async_copy(zb, agg_sh.at[pl.ds(sid * RPS, ZR)], csem).wait()
        return 0

    lax.fori_loop(0, RPS // ZR, zero_fire, 0)

    for b in range(NBUF - 1):
        pltpu.make_async_copy(ei_hbm.at[wid, b], ei_v.at[b], isems.at[b]).wait()
        pltpu.async_copy(h_hbm.at[ei_v.at[b, 0]], rows_v.at[b], gsems.at[b])

    lax.fori_loop(0, RPS // ZR, zero_drain, 0)
    plsc.subcore_barrier()

    def body(i, _):
        b = lax.rem(i, NBUF)
        qb = lax.rem(i, IB)
        pltpu.make_async_copy(h_hbm.at[ei_v.at[qb, 0]], rows_v.at[b], gsems.at[b]).wait()

        @pl.when(i >= 1)
        def _():
            bm = lax.rem(i + 2, NBUF)
            qm = lax.rem(i + 4, IB)
            pltpu.make_async_copy(
                rows_v.at[bm], agg_sh.at[ei_v.at[qm, 1]], ssems.at[bm]
            ).wait()

        @pl.when((i >= 1) & (i + 4 < NB))
        def _():
            qm = lax.rem(i + 4, IB)
            pltpu.async_copy(ei_hbm.at[wid, i + 4], ei_v.at[qm], isems.at[qm])

        @pl.when(i + 2 < NB)
        def _():
            q2 = lax.rem(i + 2, IB)
            r2 = lax.rem(i + 2, NBUF)
            pltpu.make_async_copy(ei_hbm.at[wid, i + 2], ei_v.at[q2], isems.at[q2]).wait()
            pltpu.async_copy(h_hbm.at[ei_v.at[q2, 0]], rows_v.at[r2], gsems.at[r2])

        pltpu.async_copy(rows_v.at[b], agg_sh.at[ei_v.at[qb, 1]], ssems.at[b], add=True)
        return 0

    lax.fori_loop(0, NB, body, 0)
    pltpu.make_async_copy(
        rows_v.at[(NB - 1) % NBUF],
        agg_sh.at[ei_v.at[(NB - 1) % IB, 1]],
        ssems.at[(NB - 1) % NBUF],
    ).wait()
    plsc.subcore_barrier()

    def out_fire(j, _):
        sl = pl.ds(sid * RPS + j * ZR, ZR)
        pltpu.async_copy(agg_sh.at[sl], out_hbm.at[cid, sl], csem)
        return 0

    def out_drain(j, _):
        sl = pl.ds(sid * RPS, ZR)
        pltpu.make_async_copy(agg_sh.at[sl], out_hbm.at[cid, sl], csem).wait()
        return 0

    lax.fori_loop(0, RPS // ZR, out_fire, 0)
    lax.fori_loop(0, RPS // ZR, out_drain, 0)


_RB = 2048
_GRID = NP // _RB


def _ns_nd(d):
    outd = d[:, 0:1] + d[:, 2:3]
    ind = d[:, 1:2] + d[:, 3:4]
    ns = lax.rsqrt(jnp.maximum(outd, 1.0))
    nd = lax.rsqrt(jnp.maximum(ind, 1.0))
    return ns, nd


def _scale_body(f_ref, d_ref, o_ref):
    ns, _ = _ns_nd(d_ref[...])
    o_ref[...] = f_ref[...] * ns


def _layer1_body(p_ref, d_ref, w_ref, b_ref, o_ref):
    ns, nd = _ns_nd(d_ref[...])
    rst = (p_ref[0] + p_ref[1]) * nd
    h = jnp.dot(rst, w_ref[...], preferred_element_type=jnp.float32) + b_ref[...]
    o_ref[...] = jnp.maximum(h, 0.0) * ns


def _layer2_body(p_ref, d_ref, w_ref, b_ref, o_ref):
    _, nd = _ns_nd(d_ref[...])
    rst = (p_ref[0] + p_ref[1]) * nd
    o_ref[...] = jnp.dot(rst, w_ref[...], preferred_element_type=jnp.float32) + b_ref[...]


_scale = pl.pallas_call(
    _scale_body,
    grid=(_GRID,),
    in_specs=[
        pl.BlockSpec((_RB, D), lambda i: (i, 0)),
        pl.BlockSpec((_RB, 4), lambda i: (i, 0)),
    ],
    out_specs=pl.BlockSpec((_RB, D), lambda i: (i, 0)),
    out_shape=jax.ShapeDtypeStruct((N, D), jnp.float32),
)

_layer1 = pl.pallas_call(
    _layer1_body,
    grid=(_GRID,),
    in_specs=[
        pl.BlockSpec((NC, _RB, D), lambda i: (0, i, 0)),
        pl.BlockSpec((_RB, 4), lambda i: (i, 0)),
        pl.BlockSpec((D, D), lambda i: (0, 0)),
        pl.BlockSpec((1, D), lambda i: (0, 0)),
    ],
    out_specs=pl.BlockSpec((_RB, D), lambda i: (i, 0)),
    out_shape=jax.ShapeDtypeStruct((N, D), jnp.float32),
)

_layer2 = pl.pallas_call(
    _layer2_body,
    grid=(_GRID,),
    in_specs=[
        pl.BlockSpec((NC, _RB, D), lambda i: (0, i, 0)),
        pl.BlockSpec((_RB, 4), lambda i: (i, 0)),
        pl.BlockSpec((D, D), lambda i: (0, 0)),
        pl.BlockSpec((1, D), lambda i: (0, 0)),
    ],
    out_specs=pl.BlockSpec((_RB, D), lambda i: (i, 0)),
    out_shape=jax.ShapeDtypeStruct((N, D), jnp.float32),
)


def kernel(feat, edge_index, W1, b1, W2, b2):
    ei = jnp.transpose(edge_index.reshape(2, NW, NB, K), (1, 2, 0, 3))

    degs = _deg_kernel(ei)
    degs_t = degs.reshape(2 * NC, NP).T

    h1 = _scale(feat, degs_t)
    p1 = _agg_kernel(h1, ei)
    h2 = _layer1(p1, degs_t, W1, b1.reshape(1, D))
    p2 = _agg_kernel(h2, ei)
    return _layer2(p2, degs_t, W2, b2.reshape(1, D))


# speedup vs baseline: 1.1817x; 1.0208x over previous
import functools

import jax
import jax.numpy as jnp
from jax import lax
from jax.experimental import pallas as pl
from jax.experimental.pallas import tpu as pltpu
from jax.experimental.pallas import tpu_sc as plsc

N = 10000
E = 320000
D = 128

NC = 2
NS = 16
NW = NC * NS

NP = 10240
RPS = NP // NS
EPW = E // NW
K = 80
NB = EPW // K
NBUF = 3
DNBUF = 5
ZR = 16

_mesh = plsc.VectorSubcoreMesh(core_axis_name="c", subcore_axis_name="s")


@functools.partial(
    pl.kernel,
    out_type=jax.ShapeDtypeStruct((NC, 2, NP), jnp.float32),
    mesh=_mesh,
    scratch_types=[
        pltpu.VMEM((NB, 2, K), jnp.int32),
        pltpu.VMEM((K,), jnp.float32),
        pltpu.VMEM((RPS,), jnp.float32),
        pltpu.VMEM_SHARED((NP,), jnp.float32),
        pltpu.VMEM_SHARED((NP,), jnp.float32),
        pltpu.SemaphoreType.DMA((DNBUF,)),
    ],
)
def _deg_kernel(ei_hbm, out_hbm, ei_all, ones_v, zb, do_sh, di_sh, sems):
    cid = lax.axis_index("c")
    sid = lax.axis_index("s")
    wid = cid * NS + sid

    def fill(i, _):
        zb[pl.ds(i * 16, 16)] = jnp.zeros((16,), jnp.float32)
        ones_v[pl.ds((i % (K // 16)) * 16, 16)] = jnp.ones((16,), jnp.float32)
        return 0

    lax.fori_loop(0, RPS // 16, fill, 0)
    pltpu.sync_copy(ei_hbm.at[wid], ei_all)
    pltpu.sync_copy(zb, do_sh.at[pl.ds(sid * RPS, RPS)])
    pltpu.sync_copy(zb, di_sh.at[pl.ds(sid * RPS, RPS)])
    plsc.subcore_barrier()

    for b in range(DNBUF - 1):
        pltpu.async_copy(ones_v, do_sh.at[ei_all.at[b, 0]], sems.at[b], add=True)
        pltpu.async_copy(ones_v, di_sh.at[ei_all.at[b, 1]], sems.at[b], add=True)

    def group(g, _):
        for b in range(DNBUF):
            i = g * DNBUF + b
            pltpu.make_async_copy(ones_v, do_sh.at[ei_all.at[i, 0]], sems.at[b]).wait()
            pltpu.make_async_copy(ones_v, di_sh.at[ei_all.at[i, 1]], sems.at[b]).wait()
            nxt = i + DNBUF - 1
            bx = (b + DNBUF - 1) % DNBUF

            @pl.when(nxt < NB)
            def _():
                pltpu.async_copy(ones_v, do_sh.at[ei_all.at[nxt, 0]], sems.at[bx], add=True)
                pltpu.async_copy(ones_v, di_sh.at[ei_all.at[nxt, 1]], sems.at[bx], add=True)

        return 0

    lax.fori_loop(0, NB // DNBUF, group, 0)
    plsc.subcore_barrier()

    sl = pl.ds(sid * RPS, RPS)
    pltpu.sync_copy(do_sh.at[sl], out_hbm.at[cid, 0, sl])
    pltpu.sync_copy(di_sh.at[sl], out_hbm.at[cid, 1, sl])


IB = 5


@functools.partial(
    pl.kernel,
    out_type=jax.ShapeDtypeStruct((NC, NP, D), jnp.float32),
    mesh=_mesh,
    scratch_types=[
        pltpu.VMEM((IB, 2, K), jnp.int32),
        pltpu.VMEM((NBUF, K, D), jnp.float32),
        pltpu.VMEM((ZR, D), jnp.float32),
        pltpu.VMEM_SHARED((NP, D), jnp.float32),
        pltpu.SemaphoreType.DMA((IB,)),
        pltpu.SemaphoreType.DMA((NBUF,)),
        pltpu.SemaphoreType.DMA((NBUF,)),
        pltpu.SemaphoreType.DMA,
    ],
)
def _agg_kernel(h_hbm, ei_hbm, out_hbm, ei_v, rows_v, zb, agg_sh, isems, gsems, ssems, csem):
    cid = lax.axis_index("c")
    sid = lax.axis_index("s")
    wid = cid * NS + sid

    def fill(t, _):
        zb[t // (D // 16), pl.ds((t % (D // 16)) * 16, 16)] = jnp.zeros(
            (16,), jnp.float32
        )
        return 0

    for q in range(IB):
        pltpu.async_copy(ei_hbm.at[wid, q], ei_v.at[q], isems.at[q])

    lax.fori_loop(0, ZR * (D // 16), fill, 0)

    def zero_fire(j, _):
        pltpu.async_copy(zb, agg_sh.at[pl.ds(sid * RPS + j * ZR, ZR)], csem)
        return 0

    def zero_drain(j, _):
        pltpu.make_async_copy(zb, agg_sh.at[pl.ds(sid * RPS, ZR)], csem).wait()
        return 0

    lax.fori_loop(0, RPS // ZR, zero_fire, 0)

    for b in range(NBUF - 1):
        pltpu.make_async_copy(ei_hbm.at[wid, b], ei_v.at[b], isems.at[b]).wait()
        pltpu.async_copy(h_hbm.at[ei_v.at[b, 0]], rows_v.at[b], gsems.at[b])

    lax.fori_loop(0, RPS // ZR, zero_drain, 0)
    plsc.subcore_barrier()

    def body(i, _):
        b = lax.rem(i, NBUF)
        qb = lax.rem(i, IB)
        pltpu.make_async_copy(h_hbm.at[ei_v.at[qb, 0]], rows_v.at[b], gsems.at[b]).wait()

        @pl.when(i >= 1)
        def _():
            bm = lax.rem(i + 2, NBUF)
            qm = lax.rem(i + 4, IB)
            pltpu.make_async_copy(
                rows_v.at[bm], agg_sh.at[ei_v.at[qm, 1]], ssems.at[bm]
            ).wait()

        @pl.when((i >= 1) & (i + 4 < NB))
        def _():
            qm = lax.rem(i + 4, IB)
            pltpu.async_copy(ei_hbm.at[wid, i + 4], ei_v.at[qm], isems.at[qm])

        @pl.when(i + 2 < NB)
        def _():
            q2 = lax.rem(i + 2, IB)
            r2 = lax.rem(i + 2, NBUF)
            pltpu.make_async_copy(ei_hbm.at[wid, i + 2], ei_v.at[q2], isems.at[q2]).wait()
            pltpu.async_copy(h_hbm.at[ei_v.at[q2, 0]], rows_v.at[r2], gsems.at[r2])

        pltpu.async_copy(rows_v.at[b], agg_sh.at[ei_v.at[qb, 1]], ssems.at[b], add=True)
        return 0

    lax.fori_loop(0, NB, body, 0)
    pltpu.make_async_copy(
        rows_v.at[(NB - 1) % NBUF],
        agg_sh.at[ei_v.at[(NB - 1) % IB, 1]],
        ssems.at[(NB - 1) % NBUF],
    ).wait()
    plsc.subcore_barrier()

    def out_fire(j, _):
        sl = pl.ds(sid * RPS + j * ZR, ZR)
        pltpu.async_copy(agg_sh.at[sl], out_hbm.at[cid, sl], csem)
        return 0

    def out_drain(j, _):
        sl = pl.ds(sid * RPS, ZR)
        pltpu.make_async_copy(agg_sh.at[sl], out_hbm.at[cid, sl], csem).wait()
        return 0

    lax.fori_loop(0, RPS // ZR, out_fire, 0)
    lax.fori_loop(0, RPS // ZR, out_drain, 0)


_RB = 5120
_GRID = NP // _RB


def _ns_nd(d):
    outd = d[:, 0:1] + d[:, 2:3]
    ind = d[:, 1:2] + d[:, 3:4]
    ns = lax.rsqrt(jnp.maximum(outd, 1.0))
    nd = lax.rsqrt(jnp.maximum(ind, 1.0))
    return ns, nd


def _scale_body(f_ref, d_ref, o_ref):
    ns, _ = _ns_nd(d_ref[...])
    o_ref[...] = f_ref[...] * ns


def _layer1_body(p_ref, d_ref, w_ref, b_ref, o_ref):
    ns, nd = _ns_nd(d_ref[...])
    rst = (p_ref[0] + p_ref[1]) * nd
    h = jnp.dot(rst, w_ref[...], preferred_element_type=jnp.float32) + b_ref[...]
    o_ref[...] = jnp.maximum(h, 0.0) * ns


def _layer2_body(p_ref, d_ref, w_ref, b_ref, o_ref):
    _, nd = _ns_nd(d_ref[...])
    rst = (p_ref[0] + p_ref[1]) * nd
    o_ref[...] = jnp.dot(rst, w_ref[...], preferred_element_type=jnp.float32) + b_ref[...]


_scale = pl.pallas_call(
    _scale_body,
    grid=(_GRID,),
    in_specs=[
        pl.BlockSpec((_RB, D), lambda i: (i, 0)),
        pl.BlockSpec((_RB, 4), lambda i: (i, 0)),
    ],
    out_specs=pl.BlockSpec((_RB, D), lambda i: (i, 0)),
    out_shape=jax.ShapeDtypeStruct((N, D), jnp.float32),
)

_layer1 = pl.pallas_call(
    _layer1_body,
    grid=(_GRID,),
    in_specs=[
        pl.BlockSpec((NC, _RB, D), lambda i: (0, i, 0)),
        pl.BlockSpec((_RB, 4), lambda i: (i, 0)),
        pl.BlockSpec((D, D), lambda i: (0, 0)),
        pl.BlockSpec((1, D), lambda i: (0, 0)),
    ],
    out_specs=pl.BlockSpec((_RB, D), lambda i: (i, 0)),
    out_shape=jax.ShapeDtypeStruct((N, D), jnp.float32),
)

_layer2 = pl.pallas_call(
    _layer2_body,
    grid=(_GRID,),
    in_specs=[
        pl.BlockSpec((NC, _RB, D), lambda i: (0, i, 0)),
        pl.BlockSpec((_RB, 4), lambda i: (i, 0)),
        pl.BlockSpec((D, D), lambda i: (0, 0)),
        pl.BlockSpec((1, D), lambda i: (0, 0)),
    ],
    out_specs=pl.BlockSpec((_RB, D), lambda i: (i, 0)),
    out_shape=jax.ShapeDtypeStruct((N, D), jnp.float32),
)


def kernel(feat, edge_index, W1, b1, W2, b2):
    ei = jnp.transpose(edge_index.reshape(2, NW, NB, K), (1, 2, 0, 3))

    degs = _deg_kernel(ei)
    degs_t = degs.reshape(2 * NC, NP).T

    h1 = _scale(feat, degs_t)
    p1 = _agg_kernel(h1, ei)
    h2 = _layer1(p1, degs_t, W1, b1.reshape(1, D))
    p2 = _agg_kernel(h2, ei)
    return _layer2(p2, degs_t, W2, b2.reshape(1, D))


# speedup vs baseline: 1.1829x; 1.0010x over previous
import functools

import jax
import jax.numpy as jnp
from jax import lax
from jax.experimental import pallas as pl
from jax.experimental.pallas import tpu as pltpu
from jax.experimental.pallas import tpu_sc as plsc

N = 10000
E = 320000
D = 128

NC = 2
NS = 16
NW = NC * NS

NP = 10240
RPS = NP // NS
EPW = E // NW
K = 80
NB = EPW // K
NBUF = 3
DNBUF = 5
ZR = 32

_mesh = plsc.VectorSubcoreMesh(core_axis_name="c", subcore_axis_name="s")


@functools.partial(
    pl.kernel,
    out_type=jax.ShapeDtypeStruct((NC, 2, NP), jnp.float32),
    mesh=_mesh,
    scratch_types=[
        pltpu.VMEM((NB, K), jnp.int32),
        pltpu.VMEM((NB, K), jnp.int32),
        pltpu.VMEM((K,), jnp.float32),
        pltpu.VMEM((RPS,), jnp.float32),
        pltpu.VMEM_SHARED((NP,), jnp.float32),
        pltpu.VMEM_SHARED((NP,), jnp.float32),
        pltpu.SemaphoreType.DMA((DNBUF,)),
    ],
)
def _deg_kernel(ei_hbm, out_hbm, si_all, di_all, ones_v, zb, do_sh, di_sh, sems):
    cid = lax.axis_index("c")
    sid = lax.axis_index("s")
    wid = cid * NS + sid

    def fill(i, _):
        zb[pl.ds(i * 16, 16)] = jnp.zeros((16,), jnp.float32)
        ones_v[pl.ds((i % (K // 16)) * 16, 16)] = jnp.ones((16,), jnp.float32)
        return 0

    lax.fori_loop(0, RPS // 16, fill, 0)
    pltpu.sync_copy(ei_hbm.at[0, wid], si_all)
    pltpu.sync_copy(ei_hbm.at[1, wid], di_all)
    pltpu.sync_copy(zb, do_sh.at[pl.ds(sid * RPS, RPS)])
    pltpu.sync_copy(zb, di_sh.at[pl.ds(sid * RPS, RPS)])
    plsc.subcore_barrier()

    for b in range(DNBUF - 1):
        pltpu.async_copy(ones_v, do_sh.at[si_all.at[b]], sems.at[b], add=True)
        pltpu.async_copy(ones_v, di_sh.at[di_all.at[b]], sems.at[b], add=True)

    def group(g, _):
        for b in range(DNBUF):
            i = g * DNBUF + b
            pltpu.make_async_copy(ones_v, do_sh.at[si_all.at[i]], sems.at[b]).wait()
            pltpu.make_async_copy(ones_v, di_sh.at[di_all.at[i]], sems.at[b]).wait()
            nxt = i + DNBUF - 1
            bx = (b + DNBUF - 1) % DNBUF

            @pl.when(nxt < NB)
            def _():
                pltpu.async_copy(ones_v, do_sh.at[si_all.at[nxt]], sems.at[bx], add=True)
                pltpu.async_copy(ones_v, di_sh.at[di_all.at[nxt]], sems.at[bx], add=True)

        return 0

    lax.fori_loop(0, NB // DNBUF, group, 0)
    plsc.subcore_barrier()

    sl = pl.ds(sid * RPS, RPS)
    pltpu.sync_copy(do_sh.at[sl], out_hbm.at[cid, 0, sl])
    pltpu.sync_copy(di_sh.at[sl], out_hbm.at[cid, 1, sl])


IB = 5


@functools.partial(
    pl.kernel,
    out_type=jax.ShapeDtypeStruct((NC, NP, D), jnp.float32),
    mesh=_mesh,
    scratch_types=[
        pltpu.VMEM((IB, 2, K), jnp.int32),
        pltpu.VMEM((NBUF, K, D), jnp.float32),
        pltpu.VMEM((ZR, D), jnp.float32),
        pltpu.VMEM_SHARED((NP, D), jnp.float32),
        pltpu.SemaphoreType.DMA((IB,)),
        pltpu.SemaphoreType.DMA((NBUF,)),
        pltpu.SemaphoreType.DMA((NBUF,)),
        pltpu.SemaphoreType.DMA,
    ],
)
def _agg_kernel(h_hbm, ei_hbm, out_hbm, ei_v, rows_v, zb, agg_sh, isems, gsems, ssems, csem):
    cid = lax.axis_index("c")
    sid = lax.axis_index("s")
    wid = cid * NS + sid

    def fill(t, _):
        zb[t // (D // 16), pl.ds((t % (D // 16)) * 16, 16)] = jnp.zeros(
            (16,), jnp.float32
        )
        return 0

    for q in range(IB):
        pltpu.async_copy(ei_hbm.at[wid, q], ei_v.at[q], isems.at[q])

    lax.fori_loop(0, ZR * (D // 16), fill, 0)

    def zero_fire(j, _):
        pltpu.async_copy(zb, agg_sh.at[pl.ds(sid * RPS + j * ZR, ZR)], csem)
        return 0

    def zero_drain(j, _):
        pltpu.make_async_copy(zb, agg_sh.at[pl.ds(sid * RPS, ZR)], csem).wait()
        return 0

    lax.fori_loop(0, RPS // ZR, zero_fire, 0)

    for b in range(NBUF - 1):
        pltpu.make_async_copy(ei_hbm.at[wid, b], ei_v.at[b], isems.at[b]).wait()
        pltpu.async_copy(h_hbm.at[ei_v.at[b, 0]], rows_v.at[b], gsems.at[b])

    lax.fori_loop(0, RPS // ZR, zero_drain, 0)
    plsc.subcore_barrier()

    def body(i, _):
        b = lax.rem(i, NBUF)
        qb = lax.rem(i, IB)
        pltpu.make_async_copy(h_hbm.at[ei_v.at[qb, 0]], rows_v.at[b], gsems.at[b]).wait()

        @pl.when(i >= 1)
        def _():
            bm = lax.rem(i + 2, NBUF)
            qm = lax.rem(i + 4, IB)
            pltpu.make_async_copy(
                rows_v.at[bm], agg_sh.at[ei_v.at[qm, 1]], ssems.at[bm]
            ).wait()

        @pl.when((i >= 1) & (i + 4 < NB))
        def _():
            qm = lax.rem(i + 4, IB)
            pltpu.async_copy(ei_hbm.at[wid, i + 4], ei_v.at[qm], isems.at[qm])

        @pl.when(i + 2 < NB)
        def _():
            q2 = lax.rem(i + 2, IB)
            r2 = lax.rem(i + 2, NBUF)
            pltpu.make_async_copy(ei_hbm.at[wid, i + 2], ei_v.at[q2], isems.at[q2]).wait()
            pltpu.async_copy(h_hbm.at[ei_v.at[q2, 0]], rows_v.at[r2], gsems.at[r2])

        pltpu.async_copy(rows_v.at[b], agg_sh.at[ei_v.at[qb, 1]], ssems.at[b], add=True)
        return 0

    lax.fori_loop(0, NB, body, 0)
    pltpu.make_async_copy(
        rows_v.at[(NB - 1) % NBUF],
        agg_sh.at[ei_v.at[(NB - 1) % IB, 1]],
        ssems.at[(NB - 1) % NBUF],
    ).wait()
    plsc.subcore_barrier()

    def out_fire(j, _):
        sl = pl.ds(sid * RPS + j * ZR, ZR)
        pltpu.async_copy(agg_sh.at[sl], out_hbm.at[cid, sl], csem)
        return 0

    def out_drain(j, _):
        sl = pl.ds(sid * RPS, ZR)
        pltpu.make_async_copy(agg_sh.at[sl], out_hbm.at[cid, sl], csem).wait()
        return 0

    lax.fori_loop(0, RPS // ZR, out_fire, 0)
    lax.fori_loop(0, RPS // ZR, out_drain, 0)


_RB = 5120
_GRID = NP // _RB


def _ns_nd(d):
    outd = d[:, 0:1] + d[:, 2:3]
    ind = d[:, 1:2] + d[:, 3:4]
    ns = lax.rsqrt(jnp.maximum(outd, 1.0))
    nd = lax.rsqrt(jnp.maximum(ind, 1.0))
    return ns, nd


def _scale_body(f_ref, d_ref, o_ref):
    ns, _ = _ns_nd(d_ref[...])
    o_ref[...] = f_ref[...] * ns


def _layer1_body(p_ref, d_ref, w_ref, b_ref, o_ref):
    ns, nd = _ns_nd(d_ref[...])
    rst = (p_ref[0] + p_ref[1]) * nd
    h = jnp.dot(rst, w_ref[...], preferred_element_type=jnp.float32) + b_ref[...]
    o_ref[...] = jnp.maximum(h, 0.0) * ns


def _layer2_body(p_ref, d_ref, w_ref, b_ref, o_ref):
    _, nd = _ns_nd(d_ref[...])
    rst = (p_ref[0] + p_ref[1]) * nd
    o_ref[...] = jnp.dot(rst, w_ref[...], preferred_element_type=jnp.float32) + b_ref[...]


_scale = pl.pallas_call(
    _scale_body,
    grid=(_GRID,),
    in_specs=[
        pl.BlockSpec((_RB, D), lambda i: (i, 0)),
        pl.BlockSpec((_RB, 4), lambda i: (i, 0)),
    ],
    out_specs=pl.BlockSpec((_RB, D), lambda i: (i, 0)),
    out_shape=jax.ShapeDtypeStruct((N, D), jnp.float32),
)

_layer1 = pl.pallas_call(
    _layer1_body,
    grid=(_GRID,),
    in_specs=[
        pl.BlockSpec((NC, _RB, D), lambda i: (0, i, 0)),
        pl.BlockSpec((_RB, 4), lambda i: (i, 0)),
        pl.BlockSpec((D, D), lambda i: (0, 0)),
        pl.BlockSpec((1, D), lambda i: (0, 0)),
    ],
    out_specs=pl.BlockSpec((_RB, D), lambda i: (i, 0)),
    out_shape=jax.ShapeDtypeStruct((N, D), jnp.float32),
)

_layer2 = pl.pallas_call(
    _layer2_body,
    grid=(_GRID,),
    in_specs=[
        pl.BlockSpec((NC, _RB, D), lambda i: (0, i, 0)),
        pl.BlockSpec((_RB, 4), lambda i: (i, 0)),
        pl.BlockSpec((D, D), lambda i: (0, 0)),
        pl.BlockSpec((1, D), lambda i: (0, 0)),
    ],
    out_specs=pl.BlockSpec((_RB, D), lambda i: (i, 0)),
    out_shape=jax.ShapeDtypeStruct((N, D), jnp.float32),
)


def kernel(feat, edge_index, W1, b1, W2, b2):
    ei2 = edge_index.reshape(2, NW, NB, K)
    ei = jnp.transpose(ei2, (1, 2, 0, 3))

    degs = _deg_kernel(ei2)
    degs_t = degs.reshape(2 * NC, NP).T

    h1 = _scale(feat, degs_t)
    p1 = _agg_kernel(h1, ei)
    h2 = _layer1(p1, degs_t, W1, b1.reshape(1, D))
    p2 = _agg_kernel(h2, ei)
    return _layer2(p2, degs_t, W2, b2.reshape(1, D))
